# baseline (speedup 1.0000x reference)
import functools

import jax
import jax.numpy as jnp
from jax import lax
from jax.experimental import pallas as pl
from jax.experimental.pallas import tpu as pltpu
from jax.experimental.pallas import tpu_sc as plsc

_GATHER_CHUNK = 128


def _sc_dots_kernel(nc, ns, bpw, C, D, table, vi, ctx):
    NW = nc * ns
    CH = bpw // _GATHER_CHUNK
    mesh = plsc.VectorSubcoreMesh(core_axis_name="c", subcore_axis_name="s")

    @functools.partial(
        pl.kernel,
        mesh=mesh,
        compiler_params=pltpu.CompilerParams(use_tc_tiling_on_sc=False),
        out_type=jax.ShapeDtypeStruct((NW, C, bpw), jnp.float32),
        scratch_types=[
            pltpu.VMEM((CH, _GATHER_CHUNK), jnp.int32),
            pltpu.VMEM((C, CH, _GATHER_CHUNK), jnp.int32),
            pltpu.VMEM((bpw, D), jnp.float32),
            pltpu.VMEM((bpw, D), jnp.float32),
            pltpu.VMEM((C, bpw), jnp.float32),
            pltpu.SemaphoreType.DMA,
        ],
    )
    def k(table_hbm, vi_hbm, ctx_hbm, out_hbm, vi_v, ctx_v, ei_v, cr_v, dots_v, sem):
        wid = lax.axis_index("s") * nc + lax.axis_index("c")
        pltpu.sync_copy(vi_hbm.at[wid], vi_v)
        pltpu.sync_copy(ctx_hbm.at[wid], ctx_v)

        cps = [
            pltpu.async_copy(
                table_hbm.at[vi_v.at[ch]],
                ei_v.at[pl.ds(ch * _GATHER_CHUNK, _GATHER_CHUNK)],
                sem,
            )
            for ch in range(CH)
        ]
        for cp in cps:
            cp.wait()

        lanes = lax.iota(jnp.int32, 16)
        rots = [((lanes + sh) % 16)[:, None] for sh in (8, 4, 2, 1)]
        dnums = lax.GatherDimensionNumbers(
            offset_dims=(), collapsed_slice_dims=(0,), start_index_map=(0,)
        )

        def hsum(x):
            for r in rots:
                x = x + lax.gather(
                    x, r, dnums, slice_sizes=(1,),
                    mode=lax.GatherScatterMode.PROMISE_IN_BOUNDS,
                )
            return x

        for c in range(C):
            cps = [
                pltpu.async_copy(
                    table_hbm.at[ctx_v.at[c, ch]],
                    cr_v.at[pl.ds(ch * _GATHER_CHUNK, _GATHER_CHUNK)],
                    sem,
                )
                for ch in range(CH)
            ]
            for cp in cps:
                cp.wait()

            def blk(b, _):
                dv = jnp.zeros((16,), jnp.float32)
                for u in range(16):
                    s = b * 16 + u
                    acc = ei_v[s, pl.ds(0, 16)] * cr_v[s, pl.ds(0, 16)]
                    for kk in range(1, D // 16):
                        acc = acc + (
                            ei_v[s, pl.ds(kk * 16, 16)] * cr_v[s, pl.ds(kk * 16, 16)]
                        )
                    dv = jnp.where(lanes == u, hsum(acc), dv)
                dots_v[c, pl.ds(b * 16, 16)] = dv
                return 0

            lax.fori_loop(0, bpw // 16, blk, 0, unroll=False)

        pltpu.sync_copy(dots_v, out_hbm.at[wid])

    return k(table, vi, ctx)


def _tc_loss_kernel(x2d, batch):
    def body(x_ref, o_ref):
        x = x_ref[:]
        ls = jnp.minimum(x, 0.0) - jnp.log1p(jnp.exp(-jnp.abs(x)))
        o_ref[0, 0] = -jnp.sum(ls) / batch

    return pl.pallas_call(
        body,
        out_shape=jax.ShapeDtypeStruct((1, 1), jnp.float32),
        out_specs=pl.BlockSpec(memory_space=pltpu.SMEM),
    )(x2d)


def kernel(v_i, v_j, negsamples, device, first_embeddings):
    B = v_i.shape[0]
    D = first_embeddings.shape[1]
    C = negsamples.shape[0] + 1

    info = plsc.get_sparse_core_info()
    nc, ns = info.num_cores, info.num_subcores
    NW = nc * ns
    bpw = B // NW
    CH = bpw // _GATHER_CHUNK

    vi = v_i.astype(jnp.int32).reshape(NW, CH, _GATHER_CHUNK)
    ctx = jnp.concatenate([v_j[None].astype(jnp.int32),
                           negsamples.astype(jnp.int32)], axis=0)
    ctx = ctx.reshape(C, NW, CH, _GATHER_CHUNK).transpose(1, 0, 2, 3)

    dots = _sc_dots_kernel(nc, ns, bpw, C, D, first_embeddings, vi, ctx)
    out = _tc_loss_kernel(dots.reshape(C * B // 1024, 1024), B)
    return out[0, 0]


# speedup vs baseline: 2.3954x; 2.3954x over previous
import functools

import jax
import jax.numpy as jnp
from jax import lax
from jax.experimental import pallas as pl
from jax.experimental.pallas import tpu as pltpu
from jax.experimental.pallas import tpu_sc as plsc


def _sc_dots_kernel(nc, ns, V, D, C, spt, table_t, idx_t):
    dpc = D // nc
    mesh = plsc.VectorSubcoreMesh(core_axis_name="c", subcore_axis_name="s")

    @functools.partial(
        pl.kernel,
        mesh=mesh,
        compiler_params=pltpu.CompilerParams(
            use_tc_tiling_on_sc=True, needs_layout_passes=False
        ),
        out_type=jax.ShapeDtypeStruct((nc, ns, C, spt), jnp.float32),
        scratch_types=[
            pltpu.VMEM_SHARED((V,), jnp.float32),
            pltpu.VMEM((C + 1, 1, spt), jnp.int32),
            pltpu.VMEM((C + 1, 1, spt), jnp.float32),
            pltpu.VMEM((C, 1, spt), jnp.float32),
            pltpu.SemaphoreType.DMA,
            pltpu.SemaphoreType.DMA,
            pltpu.SemaphoreType.DMA,
        ],
    )
    def k(tab, idx_h, out_h, sp0, idx_v, val_v, acc_v, semA, semB, semG):
        cid = lax.axis_index("c")
        sid = lax.axis_index("s")
        d0 = cid * dpc
        pltpu.sync_copy(idx_h.at[sid], idx_v)

        zeros16 = jnp.zeros((16,), jnp.float32)

        def zblk(b, _):
            for c in range(C):
                acc_v[c, 0, pl.ds(b * 16, 16)] = zeros16
            return 0

        lax.fori_loop(0, spt // 16, zblk, 0)

        def stage(d):
            pltpu.async_copy(tab.at[d], sp0, semA)

        def drain_row():
            pltpu.make_async_copy(tab.at[0], sp0, semA).wait()

        @pl.when(sid == 0)
        def _():
            stage(d0)

        def d_body(dl, _):
            @pl.when(sid == 0)
            def _():
                drain_row()

            plsc.subcore_barrier()

            cps = [
                pltpu.async_copy(sp0.at[idx_v.at[c, 0]], val_v.at[c, 0], semG)
                for c in range(C + 1)
            ]
            for cp in cps:
                cp.wait()

            plsc.subcore_barrier()

            @pl.when((sid == 0) & (dl + 1 < dpc))
            def _():
                stage(d0 + dl + 1)

            def blk(b, _):
                s0 = b * 16
                v0 = val_v[0, 0, pl.ds(s0, 16)]
                for c in range(C):
                    acc_v[c, 0, pl.ds(s0, 16)] = (
                        acc_v[c, 0, pl.ds(s0, 16)]
                        + v0 * val_v[c + 1, 0, pl.ds(s0, 16)]
                    )
                return 0

            lax.fori_loop(0, spt // 16, blk, 0)
            return 0

        lax.fori_loop(0, dpc, d_body, 0)

        for c in range(C):
            pltpu.sync_copy(acc_v.at[c, 0], out_h.at[cid, sid, c])

    return k(table_t, idx_t)


def _tc_loss_kernel(parts, batch):
    def body(x_ref, o_ref):
        x = x_ref[0] + x_ref[1]
        ls = jnp.minimum(x, 0.0) - jnp.log1p(jnp.exp(-jnp.abs(x)))
        o_ref[0, 0] = -jnp.sum(ls) / batch

    return pl.pallas_call(
        body,
        out_shape=jax.ShapeDtypeStruct((1, 1), jnp.float32),
        out_specs=pl.BlockSpec(memory_space=pltpu.SMEM),
    )(parts)


def kernel(v_i, v_j, negsamples, device, first_embeddings):
    B = v_i.shape[0]
    V, D = first_embeddings.shape
    C = negsamples.shape[0] + 1

    info = plsc.get_sparse_core_info()
    nc, ns = info.num_cores, info.num_subcores
    spt = B // ns

    all_idx = jnp.concatenate(
        [v_i[None].astype(jnp.int32), v_j[None].astype(jnp.int32),
         negsamples.astype(jnp.int32)], axis=0
    )
    idx_t = all_idx.reshape(C + 1, ns, 1, spt).transpose(1, 0, 2, 3)

    parts = _sc_dots_kernel(nc, ns, V, D, C, spt, first_embeddings.T, idx_t)
    out = _tc_loss_kernel(parts.reshape(nc, C * B // 1024, 1024), B)
    return out[0, 0]
